# baseline (speedup 1.0000x reference)
import functools

import jax
import jax.numpy as jnp
import numpy as np
from jax.experimental import pallas as pl
from jax.experimental.pallas import tpu as pltpu

_EPS = float(np.finfo(np.float32).tiny)
_K = 64
_TAU = 0.1
_BISECT_STEPS = 31
_ROWS_PER_BLOCK = 64
_CHUNK = 512


def _row_max(x):
    return jnp.max(x, axis=1, keepdims=True)


def _gumbel_topk_block(s_ref, g_ref, o_ref, fs_ref, khot_ref, e_ref):
    nmax = fs_ref.shape[1]
    chunks = [slice(c, c + _CHUNK) for c in range(0, nmax, _CHUNK)]

    m0 = None
    for cj in chunks:
        fsj = s_ref[:, cj] + g_ref[:, cj]
        fs_ref[:, cj] = fsj
        khot_ref[:, cj] = jnp.zeros_like(fsj)
        mj = _row_max(fsj)
        m0 = mj if m0 is None else jnp.maximum(m0, mj)

    def iteration(_, mfs):
        m = mfs / _TAU
        for cj in chunks:
            e_ref[:, cj] = jnp.exp(fs_ref[:, cj] / _TAU - m)
        s = jnp.sum(e_ref[...], axis=1, keepdims=True)
        mnew = None
        for cj in chunks:
            onej = e_ref[:, cj] / s
            khot_ref[:, cj] = khot_ref[:, cj] + onej
            fsj = fs_ref[:, cj] + jnp.log(jnp.maximum(1.0 - onej, _EPS))
            fs_ref[:, cj] = fsj
            mj = _row_max(fsj)
            mnew = mj if mnew is None else jnp.maximum(mnew, mj)
        return mnew

    jax.lax.fori_loop(0, _K, iteration, m0, unroll=2)

    rows = fs_ref.shape[0]
    lo = jnp.zeros((rows, 1), jnp.int32)
    hi = jnp.full((rows, 1), 0x43000000, jnp.int32)
    cnt_lo = jnp.full((rows, 1), float(nmax), jnp.float32)

    def bisect_cond(carry):
        step, _, _, cnt_lo = carry
        return jnp.logical_and(step < _BISECT_STEPS, jnp.any(cnt_lo != _K))

    def bisect(carry):
        step, lo, hi, cnt_lo = carry
        mid = (lo + hi) // 2
        thr = jax.lax.bitcast_convert_type(mid, jnp.float32)
        cnt = None
        for cj in chunks:
            cj_cnt = jnp.sum(
                jnp.where(khot_ref[:, cj] >= thr, 1.0, 0.0), axis=1, keepdims=True
            )
            cnt = cj_cnt if cnt is None else cnt + cj_cnt
        ge = cnt >= _K
        return (
            step + 1,
            jnp.where(ge, mid, lo),
            jnp.where(ge, hi, mid),
            jnp.where(ge, cnt, cnt_lo),
        )

    _, lo, hi, _ = jax.lax.while_loop(bisect_cond, bisect, (0, lo, hi, cnt_lo))
    v64 = jax.lax.bitcast_convert_type(lo, jnp.float32)
    for cj in chunks:
        khotj = khot_ref[:, cj]
        hardj = jnp.where(khotj >= v64, 1.0, 0.0)
        o_ref[:, cj] = (hardj - khotj) + khotj


def kernel(scores, train_ensemble, gumbel):
    bsz, Nmax, ensemble = scores.shape
    te = gumbel.shape[0] // (bsz * ensemble)
    flat_scores = scores.reshape(bsz * ensemble, Nmax)
    r = _ROWS_PER_BLOCK
    out = pl.pallas_call(
        _gumbel_topk_block,
        grid=(bsz * ensemble * te // r,),
        in_specs=[
            pl.BlockSpec((r, Nmax), lambda i: (i, 0)),
            pl.BlockSpec((r, Nmax), lambda i: (i, 0)),
        ],
        out_specs=pl.BlockSpec((r, Nmax), lambda i: (i, 0)),
        out_shape=jax.ShapeDtypeStruct((te * bsz * ensemble, Nmax), jnp.float32),
        scratch_shapes=[
            pltpu.VMEM((r, Nmax), jnp.float32),
            pltpu.VMEM((r, Nmax), jnp.float32),
            pltpu.VMEM((r, Nmax), jnp.float32),
        ],
    )(flat_scores, gumbel)
    return out.reshape(te, bsz, ensemble, Nmax).transpose(0, 1, 3, 2)


# speedup vs baseline: 1.0657x; 1.0657x over previous
import functools

import jax
import jax.numpy as jnp
import numpy as np
from jax.experimental import pallas as pl
from jax.experimental.pallas import tpu as pltpu

_EPS = float(np.finfo(np.float32).tiny)
_K = 64
_TAU = 0.1
_BISECT_STEPS = 31
_ROWS_PER_BLOCK = 64
_CHUNK = 512


def _row_max(x):
    return jnp.max(x, axis=1, keepdims=True)


def _gumbel_topk_block(s_ref, g_ref, o_ref, fs_ref, khot_ref, e_ref):
    nmax = fs_ref.shape[1]
    chunks = [slice(c, c + _CHUNK) for c in range(0, nmax, _CHUNK)]

    rows_total = fs_ref.shape[0]
    halves = [slice(0, rows_total // 2), slice(rows_total // 2, rows_total)]

    m0s = []
    for rs in halves:
        m0 = None
        for cj in chunks:
            fsj = s_ref[rs, cj] + g_ref[rs, cj]
            fs_ref[rs, cj] = fsj
            khot_ref[rs, cj] = jnp.zeros_like(fsj)
            mj = _row_max(fsj)
            m0 = mj if m0 is None else jnp.maximum(m0, mj)
        m0s.append(m0)

    def half_iteration(rs, mfs):
        m = mfs / _TAU
        for cj in chunks:
            e_ref[rs, cj] = jnp.exp(fs_ref[rs, cj] / _TAU - m)
        s = jnp.sum(e_ref[rs, :], axis=1, keepdims=True)
        mnew = None
        for cj in chunks:
            onej = e_ref[rs, cj] / s
            khot_ref[rs, cj] = khot_ref[rs, cj] + onej
            fsj = fs_ref[rs, cj] + jnp.log(jnp.maximum(1.0 - onej, _EPS))
            fs_ref[rs, cj] = fsj
            mj = _row_max(fsj)
            mnew = mj if mnew is None else jnp.maximum(mnew, mj)
        return mnew

    def iteration(_, mfss):
        return tuple(half_iteration(rs, m) for rs, m in zip(halves, mfss))

    jax.lax.fori_loop(0, _K, iteration, tuple(m0s))

    rows = fs_ref.shape[0]
    lo = jnp.zeros((rows, 1), jnp.int32)
    hi = jnp.full((rows, 1), 0x43000000, jnp.int32)
    cnt_lo = jnp.full((rows, 1), float(nmax), jnp.float32)

    def bisect_cond(carry):
        step, _, _, cnt_lo = carry
        return jnp.logical_and(step < _BISECT_STEPS, jnp.any(cnt_lo != _K))

    def bisect(carry):
        step, lo, hi, cnt_lo = carry
        mid = (lo + hi) // 2
        thr = jax.lax.bitcast_convert_type(mid, jnp.float32)
        cnt = None
        for cj in chunks:
            cj_cnt = jnp.sum(
                jnp.where(khot_ref[:, cj] >= thr, 1.0, 0.0), axis=1, keepdims=True
            )
            cnt = cj_cnt if cnt is None else cnt + cj_cnt
        ge = cnt >= _K
        return (
            step + 1,
            jnp.where(ge, mid, lo),
            jnp.where(ge, hi, mid),
            jnp.where(ge, cnt, cnt_lo),
        )

    _, lo, hi, _ = jax.lax.while_loop(bisect_cond, bisect, (0, lo, hi, cnt_lo))
    v64 = jax.lax.bitcast_convert_type(lo, jnp.float32)
    for cj in chunks:
        khotj = khot_ref[:, cj]
        hardj = jnp.where(khotj >= v64, 1.0, 0.0)
        o_ref[:, cj] = (hardj - khotj) + khotj


def kernel(scores, train_ensemble, gumbel):
    bsz, Nmax, ensemble = scores.shape
    te = gumbel.shape[0] // (bsz * ensemble)
    flat_scores = scores.reshape(bsz * ensemble, Nmax)
    r = _ROWS_PER_BLOCK
    out = pl.pallas_call(
        _gumbel_topk_block,
        grid=(bsz * ensemble * te // r,),
        in_specs=[
            pl.BlockSpec((r, Nmax), lambda i: (i, 0)),
            pl.BlockSpec((r, Nmax), lambda i: (i, 0)),
        ],
        out_specs=pl.BlockSpec((r, Nmax), lambda i: (i, 0)),
        out_shape=jax.ShapeDtypeStruct((te * bsz * ensemble, Nmax), jnp.float32),
        scratch_shapes=[
            pltpu.VMEM((r, Nmax), jnp.float32),
            pltpu.VMEM((r, Nmax), jnp.float32),
            pltpu.VMEM((r, Nmax), jnp.float32),
        ],
    )(flat_scores, gumbel)
    return out.reshape(te, bsz, ensemble, Nmax).transpose(0, 1, 3, 2)


# speedup vs baseline: 1.0679x; 1.0020x over previous
import functools

import jax
import jax.numpy as jnp
import numpy as np
from jax.experimental import pallas as pl
from jax.experimental.pallas import tpu as pltpu

_EPS = float(np.finfo(np.float32).tiny)
_K = 64
_TAU = 0.1
_BISECT_STEPS = 31
_ROWS_PER_BLOCK = 64
_CHUNK = 512


def _row_max(x):
    return jnp.max(x, axis=1, keepdims=True)


def _gumbel_topk_block(s_ref, g_ref, o_ref, fs_ref, khot_ref, e_ref):
    nmax = fs_ref.shape[1]
    chunks = [slice(c, c + _CHUNK) for c in range(0, nmax, _CHUNK)]

    m0 = None
    for cj in chunks:
        fsj = s_ref[:, cj] + g_ref[:, cj]
        fs_ref[:, cj] = fsj
        khot_ref[:, cj] = jnp.zeros_like(fsj)
        mj = _row_max(fsj)
        m0 = mj if m0 is None else jnp.maximum(m0, mj)

    def iteration(_, mfs):
        m = mfs / _TAU
        for cj in chunks:
            e_ref[:, cj] = jnp.exp(fs_ref[:, cj] / _TAU - m)
        s = jnp.sum(e_ref[...], axis=1, keepdims=True)
        mnew = None
        for cj in chunks:
            onej = e_ref[:, cj] / s
            khot_ref[:, cj] = khot_ref[:, cj] + onej
            fsj = fs_ref[:, cj] + jnp.log(jnp.maximum(1.0 - onej, _EPS))
            fs_ref[:, cj] = fsj
            mj = _row_max(fsj)
            mnew = mj if mnew is None else jnp.maximum(mnew, mj)
        return mnew

    jax.lax.fori_loop(0, _K, iteration, m0)

    rows = fs_ref.shape[0]
    lo = jnp.zeros((rows, 1), jnp.int32)
    hi = jnp.full((rows, 1), 0x43000000, jnp.int32)
    cnt_lo = jnp.full((rows, 1), float(nmax), jnp.float32)

    def bisect_cond(carry):
        step, _, _, cnt_lo = carry
        return jnp.logical_and(step < _BISECT_STEPS, jnp.any(cnt_lo != _K))

    def bisect(carry):
        step, lo, hi, cnt_lo = carry
        mid = (lo + hi) // 2
        thr = jax.lax.bitcast_convert_type(mid, jnp.float32)
        cnt = None
        for cj in chunks:
            cj_cnt = jnp.sum(
                jnp.where(khot_ref[:, cj] >= thr, 1.0, 0.0), axis=1, keepdims=True
            )
            cnt = cj_cnt if cnt is None else cnt + cj_cnt
        ge = cnt >= _K
        return (
            step + 1,
            jnp.where(ge, mid, lo),
            jnp.where(ge, hi, mid),
            jnp.where(ge, cnt, cnt_lo),
        )

    _, lo, hi, _ = jax.lax.while_loop(bisect_cond, bisect, (0, lo, hi, cnt_lo))
    v64 = jax.lax.bitcast_convert_type(lo, jnp.float32)
    for cj in chunks:
        khotj = khot_ref[:, cj]
        hardj = jnp.where(khotj >= v64, 1.0, 0.0)
        o_ref[:, cj] = (hardj - khotj) + khotj


def kernel(scores, train_ensemble, gumbel):
    bsz, Nmax, ensemble = scores.shape
    te = gumbel.shape[0] // (bsz * ensemble)
    flat_scores = scores.reshape(bsz * ensemble, Nmax)
    r = _ROWS_PER_BLOCK
    out = pl.pallas_call(
        _gumbel_topk_block,
        grid=(bsz * ensemble * te // r,),
        in_specs=[
            pl.BlockSpec((r, Nmax), lambda i: (i, 0)),
            pl.BlockSpec((r, Nmax), lambda i: (i, 0)),
        ],
        out_specs=pl.BlockSpec((r, Nmax), lambda i: (i, 0)),
        out_shape=jax.ShapeDtypeStruct((te * bsz * ensemble, Nmax), jnp.float32),
        scratch_shapes=[
            pltpu.VMEM((r, Nmax), jnp.float32),
            pltpu.VMEM((r, Nmax), jnp.float32),
            pltpu.VMEM((r, Nmax), jnp.float32),
        ],
    )(flat_scores, gumbel)
    return out.reshape(te, bsz, ensemble, Nmax).transpose(0, 1, 3, 2)


# speedup vs baseline: 1.0774x; 1.0090x over previous
import functools

import jax
import jax.numpy as jnp
import numpy as np
from jax.experimental import pallas as pl
from jax.experimental.pallas import tpu as pltpu

_EPS = float(np.finfo(np.float32).tiny)
_K = 64
_TAU = 0.1
_BISECT_STEPS = 31
_ROWS_PER_BLOCK = 64
_CHUNK = 512


def _row_max(x):
    return jnp.max(x, axis=1, keepdims=True)


def _gumbel_topk_block(s_ref, g_ref, o_ref, fs_ref, khot_ref, e_ref):
    nmax = fs_ref.shape[1]
    chunks = [slice(c, c + _CHUNK) for c in range(0, nmax, _CHUNK)]

    m0 = None
    for cj in chunks:
        fsj = s_ref[:, cj] + g_ref[:, cj]
        fs_ref[:, cj] = fsj
        khot_ref[:, cj] = jnp.zeros_like(fsj)
        mj = _row_max(fsj)
        m0 = mj if m0 is None else jnp.maximum(m0, mj)

    def iteration(_, mfs):
        m = mfs / _TAU
        stack = []
        for cj in chunks:
            ej = jnp.exp(fs_ref[:, cj] / _TAU - m)
            e_ref[:, cj] = ej
            cols = [ej[:, c : c + 128] for c in range(0, _CHUNK, 128)]
            while len(cols) > 1:
                cols = [cols[i] + cols[i + 1] for i in range(0, len(cols), 2)]
            part = (0, cols[0])
            while stack and stack[-1][0] == part[0]:
                lvl, q = stack.pop()
                part = (lvl + 1, q + part[1])
            stack.append(part)
        assert len(stack) == 1
        s = jnp.sum(stack[0][1], axis=1, keepdims=True)
        mnew = None
        for cj in chunks:
            onej = e_ref[:, cj] / s
            khot_ref[:, cj] = khot_ref[:, cj] + onej
            fsj = fs_ref[:, cj] + jnp.log(jnp.maximum(1.0 - onej, _EPS))
            fs_ref[:, cj] = fsj
            mj = _row_max(fsj)
            mnew = mj if mnew is None else jnp.maximum(mnew, mj)
        return mnew

    jax.lax.fori_loop(0, _K, iteration, m0)

    rows = fs_ref.shape[0]
    lo = jnp.zeros((rows, 1), jnp.int32)
    hi = jnp.full((rows, 1), 0x43000000, jnp.int32)
    cnt_lo = jnp.full((rows, 1), float(nmax), jnp.float32)

    def bisect_cond(carry):
        step, _, _, cnt_lo = carry
        return jnp.logical_and(step < _BISECT_STEPS, jnp.any(cnt_lo != _K))

    def bisect(carry):
        step, lo, hi, cnt_lo = carry
        mid = (lo + hi) // 2
        thr = jax.lax.bitcast_convert_type(mid, jnp.float32)
        cnt = None
        for cj in chunks:
            cj_cnt = jnp.sum(
                jnp.where(khot_ref[:, cj] >= thr, 1.0, 0.0), axis=1, keepdims=True
            )
            cnt = cj_cnt if cnt is None else cnt + cj_cnt
        ge = cnt >= _K
        return (
            step + 1,
            jnp.where(ge, mid, lo),
            jnp.where(ge, hi, mid),
            jnp.where(ge, cnt, cnt_lo),
        )

    _, lo, hi, _ = jax.lax.while_loop(bisect_cond, bisect, (0, lo, hi, cnt_lo))
    v64 = jax.lax.bitcast_convert_type(lo, jnp.float32)
    for cj in chunks:
        khotj = khot_ref[:, cj]
        hardj = jnp.where(khotj >= v64, 1.0, 0.0)
        o_ref[:, cj] = (hardj - khotj) + khotj


def kernel(scores, train_ensemble, gumbel):
    bsz, Nmax, ensemble = scores.shape
    te = gumbel.shape[0] // (bsz * ensemble)
    flat_scores = scores.reshape(bsz * ensemble, Nmax)
    r = _ROWS_PER_BLOCK
    out = pl.pallas_call(
        _gumbel_topk_block,
        grid=(bsz * ensemble * te // r,),
        in_specs=[
            pl.BlockSpec((r, Nmax), lambda i: (i, 0)),
            pl.BlockSpec((r, Nmax), lambda i: (i, 0)),
        ],
        out_specs=pl.BlockSpec((r, Nmax), lambda i: (i, 0)),
        out_shape=jax.ShapeDtypeStruct((te * bsz * ensemble, Nmax), jnp.float32),
        scratch_shapes=[
            pltpu.VMEM((r, Nmax), jnp.float32),
            pltpu.VMEM((r, Nmax), jnp.float32),
            pltpu.VMEM((r, Nmax), jnp.float32),
        ],
    )(flat_scores, gumbel)
    return out.reshape(te, bsz, ensemble, Nmax).transpose(0, 1, 3, 2)
